# Initial kernel scaffold; baseline (speedup 1.0000x reference)
import functools

import jax
import jax.numpy as jnp
from jax import lax
from jax.experimental import pallas as pl
from jax.experimental.pallas import tpu as pltpu
from jax.experimental.pallas import tpu_sc as plsc

N = 10000
E = 320000
F_IN = 128
H = 128
C = 64

NW = 32
L = 16
CH = 81
EPW = CH * 128
EPAD = NW * EPW
TRASH = N
NPAD = 10240
RB = 1000
NB = 1024

_mesh = plsc.VectorSubcoreMesh(core_axis_name="c", subcore_axis_name="s")


def _wid():
    return lax.axis_index("s") * 2 + lax.axis_index("c")


def _lane_shift(v, k, fill):
    idx = jnp.maximum(lax.iota(jnp.int32, L) - k, 0)
    shifted = jnp.take(v, idx, mode="promise_in_bounds")
    return jnp.where(lax.iota(jnp.int32, L) >= k, shifted, fill)


def _seg_tail_reduce(keys, vals, is_sum):
    ident = jnp.float32(0.0) if is_sum else jnp.float32(-1e30)
    ks, vs = plsc.sort_key_val(keys, vals)
    for k in (1, 2, 4, 8):
        pk = _lane_shift(ks, k, jnp.int32(-1))
        pv = _lane_shift(vs, k, ident)
        same = pk == ks
        if is_sum:
            vs = vs + jnp.where(same, pv, ident)
        else:
            vs = jnp.maximum(vs, jnp.where(same, pv, ident))
    nxt = jnp.take(ks, jnp.minimum(lax.iota(jnp.int32, L) + 1, L - 1),
                   mode="promise_in_bounds")
    tail = (lax.iota(jnp.int32, L) == L - 1) | (nxt != ks)
    return ks, vs, tail


def _seg_sum_update(acc_ref, keys, vals):
    ks, vs, tail = _seg_tail_reduce(keys, vals, is_sum=True)
    plsc.addupdate_scatter(acc_ref, [ks], vs, mask=tail)


def _seg_max_update(acc_ref, keys, vals):
    ks, vs, tail = _seg_tail_reduce(keys, vals, is_sum=False)
    cur = plsc.load_gather(acc_ref, [ks], mask=tail)
    plsc.store_scatter(acc_ref, [ks], jnp.maximum(cur, vs), mask=tail)


def _leaky(t):
    return jnp.where(t >= 0, t, 0.2 * t)


@functools.partial(
    pl.kernel, mesh=_mesh,
    out_type=jax.ShapeDtypeStruct((NW, NPAD), jnp.float32),
    scratch_types=[
        pltpu.VMEM((EPW,), jnp.int32),
        pltpu.VMEM((NPAD,), jnp.float32),
        pltpu.SemaphoreType.DMA,
    ],
)
def _sc_deg(dst_hbm, out_hbm, dst_v, acc, sem):
    wid = _wid()
    pltpu.sync_copy(dst_hbm.at[wid], dst_v)

    @pl.loop(0, NPAD, step=L)
    def _(i):
        acc[pl.ds(i, L)] = jnp.zeros((L,), jnp.float32)

    ones = jnp.ones((L,), jnp.float32)

    @pl.loop(0, EPW, step=L)
    def _(i):
        _seg_sum_update(acc, dst_v[pl.ds(i, L)], ones)

    pltpu.sync_copy(acc, out_hbm.at[wid])


@functools.partial(
    pl.kernel, mesh=_mesh,
    out_type=jax.ShapeDtypeStruct((NW, NPAD), jnp.float32),
    scratch_types=[
        pltpu.VMEM((EPW,), jnp.int32),
        pltpu.VMEM((EPW,), jnp.int32),
        pltpu.VMEM((NPAD,), jnp.float32),
        pltpu.VMEM((NPAD,), jnp.float32),
        pltpu.VMEM((NPAD,), jnp.float32),
        pltpu.SemaphoreType.DMA,
    ],
)
def _sc_segmax(src_hbm, dst_hbm, s_hbm, d_hbm, out_hbm,
               src_v, dst_v, s_v, d_v, acc, sem):
    wid = _wid()
    pltpu.sync_copy(src_hbm.at[wid], src_v)
    pltpu.sync_copy(dst_hbm.at[wid], dst_v)
    pltpu.sync_copy(s_hbm, s_v)
    pltpu.sync_copy(d_hbm, d_v)

    @pl.loop(0, NPAD, step=L)
    def _(i):
        acc[pl.ds(i, L)] = jnp.full((L,), -1e30, jnp.float32)

    @pl.loop(0, EPW, step=L)
    def _(i):
        src16 = src_v[pl.ds(i, L)]
        dst16 = dst_v[pl.ds(i, L)]
        alpha = _leaky(plsc.load_gather(s_v, [src16])
                       + plsc.load_gather(d_v, [dst16]))
        _seg_max_update(acc, dst16, alpha)

    pltpu.sync_copy(acc, out_hbm.at[wid])


@functools.partial(
    pl.kernel, mesh=_mesh,
    out_type=jax.ShapeDtypeStruct((2, NPAD, H), jnp.float32),
    scratch_types=[
        pltpu.VMEM((CH, 128), jnp.int32),
        pltpu.VMEM((CH, 128), jnp.int32),
        pltpu.VMEM((128, H), jnp.float32),
        pltpu.VMEM_SHARED((NPAD, H), jnp.float32),
        pltpu.SemaphoreType.DMA,
    ],
)
def _sc_gcn_rows(src_hbm, dst_hbm, tab_hbm, out_hbm,
                 src_v, dst_v, rows, acc_sh, sem):
    core = lax.axis_index("c")
    sid = lax.axis_index("s")
    wid = _wid()
    pltpu.sync_copy(src_hbm.at[wid], src_v)
    pltpu.sync_copy(dst_hbm.at[wid], dst_v)

    @pl.loop(0, 128)
    def _(r):
        for c in range(H // L):
            rows[r, pl.ds(c * L, L)] = jnp.zeros((L,), jnp.float32)

    base = sid * (NPAD // 16)
    for k in range(NPAD // 16 // 128):
        pltpu.sync_copy(rows, acc_sh.at[pl.ds(base + k * 128, 128)])
    plsc.subcore_barrier()

    @pl.loop(0, CH)
    def _(i):
        pltpu.async_copy(tab_hbm.at[src_v.at[i]], rows, sem).wait()
        pltpu.sync_copy(rows, acc_sh.at[dst_v.at[i]], add=True)

    plsc.subcore_barrier()
    for k in range(NPAD // 16 // 128):
        sl = pl.ds(base + k * 128, 128)
        pltpu.sync_copy(acc_sh.at[sl], out_hbm.at[core, sl])


@functools.partial(
    pl.kernel, mesh=_mesh,
    out_type=(jax.ShapeDtypeStruct((2, NPAD, H), jnp.float32),
              jax.ShapeDtypeStruct((NW, NPAD), jnp.float32)),
    scratch_types=[
        pltpu.VMEM((CH, 128), jnp.int32),
        pltpu.VMEM((CH, 128), jnp.int32),
        pltpu.VMEM((NPAD,), jnp.float32),
        pltpu.VMEM((NPAD,), jnp.float32),
        pltpu.VMEM((NPAD,), jnp.float32),
        pltpu.VMEM((NPAD,), jnp.float32),
        pltpu.VMEM((128,), jnp.float32),
        pltpu.VMEM((128, H), jnp.float32),
        pltpu.VMEM_SHARED((NPAD, H), jnp.float32),
        pltpu.SemaphoreType.DMA,
    ],
)
def _sc_gat_rows(src_hbm, dst_hbm, tab_hbm, s_hbm, d_hbm, m_hbm,
                 out_hbm, den_hbm,
                 src_v, dst_v, s_v, d_v, m_v, den_acc, w_buf, rows,
                 acc_sh, sem):
    core = lax.axis_index("c")
    sid = lax.axis_index("s")
    wid = _wid()
    pltpu.sync_copy(src_hbm.at[wid], src_v)
    pltpu.sync_copy(dst_hbm.at[wid], dst_v)
    pltpu.sync_copy(s_hbm, s_v)
    pltpu.sync_copy(d_hbm, d_v)
    pltpu.sync_copy(m_hbm, m_v)

    @pl.loop(0, NPAD, step=L)
    def _(i):
        den_acc[pl.ds(i, L)] = jnp.zeros((L,), jnp.float32)

    @pl.loop(0, 128)
    def _(r):
        for c in range(H // L):
            rows[r, pl.ds(c * L, L)] = jnp.zeros((L,), jnp.float32)

    base = sid * (NPAD // 16)
    for k in range(NPAD // 16 // 128):
        pltpu.sync_copy(rows, acc_sh.at[pl.ds(base + k * 128, 128)])
    plsc.subcore_barrier()

    @pl.loop(0, CH)
    def _(i):
        cp = pltpu.async_copy(tab_hbm.at[src_v.at[i]], rows, sem)
        for j in range(8):
            src16 = src_v[i, pl.ds(j * L, L)]
            dst16 = dst_v[i, pl.ds(j * L, L)]
            alpha = _leaky(plsc.load_gather(s_v, [src16])
                           + plsc.load_gather(d_v, [dst16]))
            ex = jnp.exp(alpha - plsc.load_gather(m_v, [dst16]))
            w_buf[pl.ds(j * L, L)] = ex
            _seg_sum_update(den_acc, dst16, ex)
        cp.wait()

        @pl.loop(0, 128)
        def _(r):
            w = w_buf[r]
            for c in range(H // L):
                rows[r, pl.ds(c * L, L)] = rows[r, pl.ds(c * L, L)] * w

        pltpu.sync_copy(rows, acc_sh.at[dst_v.at[i]], add=True)

    plsc.subcore_barrier()
    for k in range(NPAD // 16 // 128):
        sl = pl.ds(base + k * 128, 128)
        pltpu.sync_copy(acc_sh.at[sl], out_hbm.at[core, sl])
    pltpu.sync_copy(den_acc, den_hbm.at[wid])


def _row_spec(b, cols=None):
    if cols is None:
        return pl.BlockSpec((b,), lambda i: (i,))
    return pl.BlockSpec((b, cols), lambda i: (i, 0))


def _full_spec(shape):
    nd = len(shape)
    return pl.BlockSpec(shape, lambda i: (0,) * nd)


def _tc_pre(x_pad, Wg0, degP):
    def body(x_ref, w_ref, deg_ref, g_ref, dinv_ref):
        deg = jnp.sum(deg_ref[...], axis=0)
        dinv = jnp.where(deg > 0, lax.rsqrt(deg), 0.0)
        hw = jnp.dot(x_ref[...], w_ref[...],
                     preferred_element_type=jnp.float32)
        g_ref[...] = hw * dinv[:, None]
        dinv_ref[...] = dinv

    return pl.pallas_call(
        body,
        grid=(NPAD // NB,),
        in_specs=[_row_spec(NB, F_IN), _full_spec((F_IN, H)),
                  pl.BlockSpec((NW, NB), lambda i: (0, i))],
        out_specs=[_row_spec(NB, H), _row_spec(NB)],
        out_shape=[jax.ShapeDtypeStruct((NPAD, H), jnp.float32),
                   jax.ShapeDtypeStruct((NPAD,), jnp.float32)],
    )(x_pad, Wg0, degP)


def _tc_gcn_mid(rowP, dinv, b, W, scale_out):
    def body(p_ref, dinv_ref, b_ref, w_ref, g_ref):
        agg = p_ref[0] + p_ref[1]
        dinv_v = dinv_ref[...]
        h = jnp.maximum(agg * dinv_v[:, None] + b_ref[...][None, :], 0.0)
        hw = jnp.dot(h, w_ref[...], preferred_element_type=jnp.float32)
        g_ref[...] = hw * dinv_v[:, None] if scale_out else hw

    return pl.pallas_call(
        body,
        grid=(NPAD // NB,),
        in_specs=[pl.BlockSpec((2, NB, H), lambda i: (0, i, 0)),
                  _row_spec(NB), _full_spec((H,)), _full_spec((H, H))],
        out_specs=_row_spec(NB, H),
        out_shape=jax.ShapeDtypeStruct((NPAD, H), jnp.float32),
    )(rowP, dinv, b, W)


def _tc_logits(hw, a_s, a_d):
    def body(hw_ref, as_ref, ad_ref, s_ref, d_ref):
        hwv = hw_ref[...]
        s_ref[...] = jnp.sum(hwv * as_ref[...][None, :], axis=1)
        d_ref[...] = jnp.sum(hwv * ad_ref[...][None, :], axis=1)

    return pl.pallas_call(
        body,
        grid=(NPAD // NB,),
        in_specs=[_row_spec(NB, H), _full_spec((H,)), _full_spec((H,))],
        out_specs=[_row_spec(NB), _row_spec(NB)],
        out_shape=[jax.ShapeDtypeStruct((NPAD,), jnp.float32),
                   jax.ShapeDtypeStruct((NPAD,), jnp.float32)],
    )(hw, a_s, a_d)


def _tc_max_combine(mP):
    def body(m_ref, o_ref):
        o_ref[...] = jnp.max(m_ref[...], axis=0)

    return pl.pallas_call(
        body,
        grid=(NPAD // NB,),
        in_specs=[pl.BlockSpec((NW, NB), lambda i: (0, i))],
        out_specs=_row_spec(NB),
        out_shape=jax.ShapeDtypeStruct((NPAD,), jnp.float32),
    )(mP)


def _tc_gat_mid(rowP, denP, b, W):
    def body(p_ref, den_ref, b_ref, w_ref, g_ref):
        denom = jnp.sum(den_ref[...], axis=0)
        agg = (p_ref[0] + p_ref[1]) / (denom + 1e-16)[:, None]
        h = jnp.maximum(agg + b_ref[...][None, :], 0.0)
        g_ref[...] = jnp.dot(h, w_ref[...],
                             preferred_element_type=jnp.float32)

    return pl.pallas_call(
        body,
        grid=(NPAD // NB,),
        in_specs=[pl.BlockSpec((2, NB, H), lambda i: (0, i, 0)),
                  pl.BlockSpec((NW, NB), lambda i: (0, i)),
                  _full_spec((H,)), _full_spec((H, H))],
        out_specs=_row_spec(NB, H),
        out_shape=jax.ShapeDtypeStruct((NPAD, H), jnp.float32),
    )(rowP, denP, b, W)


def _tc_final(rowP, denP, ba, Wl, bl):
    def body(p_ref, den_ref, ba_ref, w_ref, bl_ref, o_ref):
        denom = jnp.sum(den_ref[...], axis=0)
        agg = (p_ref[0] + p_ref[1]) / (denom + 1e-16)[:, None]
        h = jnp.maximum(agg + ba_ref[...][None, :], 0.0)
        o_ref[...] = (jnp.dot(h, w_ref[...],
                              preferred_element_type=jnp.float32)
                      + bl_ref[...][None, :])

    return pl.pallas_call(
        body,
        grid=(N // RB,),
        in_specs=[pl.BlockSpec((2, RB, H), lambda i: (0, i, 0)),
                  pl.BlockSpec((NW, RB), lambda i: (0, i)),
                  _full_spec((H,)), _full_spec((H, C)), _full_spec((C,))],
        out_specs=pl.BlockSpec((RB, C), lambda i: (i, 0)),
        out_shape=jax.ShapeDtypeStruct((N, C), jnp.float32),
    )(rowP, denP, ba, Wl, bl)


def kernel(x, edge_index, Wg0, bg0, Wg1, bg1, Wa0, as0, ad0, ba0,
           Wa1, as1, ad1, ba1, Wl, bl):
    padn = EPAD - (E + N)
    loop = jnp.arange(N, dtype=jnp.int32)
    src = jnp.concatenate(
        [edge_index[0], loop, jnp.arange(padn, dtype=jnp.int32) % N])
    dst = jnp.concatenate(
        [edge_index[1], loop, jnp.full((padn,), TRASH, jnp.int32)])
    src2 = src.reshape(NW, EPW)
    dst2 = dst.reshape(NW, EPW)
    src3 = src.reshape(NW, CH, 128)
    dst3 = dst.reshape(NW, CH, 128)
    x_pad = jnp.pad(x, ((0, NPAD - N), (0, 0)))

    degP = _sc_deg(dst2)

    g0, dinv = _tc_pre(x_pad, Wg0, degP)
    rowP1 = _sc_gcn_rows(src3, dst3, g0)
    g1 = _tc_gcn_mid(rowP1, dinv, bg0, Wg1, scale_out=True)
    rowP2 = _sc_gcn_rows(src3, dst3, g1)
    hw2 = _tc_gcn_mid(rowP2, dinv, bg1, Wa0, scale_out=False)
    s2, d2 = _tc_logits(hw2, as0, ad0)
    m2 = _tc_max_combine(_sc_segmax(src2, dst2, s2, d2))
    rowP3, denP3 = _sc_gat_rows(src3, dst3, hw2, s2, d2, m2)
    hw3 = _tc_gat_mid(rowP3, denP3, ba0, Wa1)
    s3, d3 = _tc_logits(hw3, as1, ad1)
    m3 = _tc_max_combine(_sc_segmax(src2, dst2, s3, d3))
    rowP4, denP4 = _sc_gat_rows(src3, dst3, hw3, s3, d3, m3)
    return _tc_final(rowP4, denP4, ba1, Wl, bl)


# speedup vs baseline: 15.9221x; 15.9221x over previous
import dataclasses
import functools

import jax
import jax.numpy as jnp
from jax import lax
from jax.experimental import pallas as pl
from jax.experimental.pallas import tpu as pltpu
from jax.experimental.pallas import tpu_sc as plsc

N = 10000
E = 320000
F_IN = 128
H = 128
C = 64

NW = 32
L = 16
CH = 81
EPW = CH * 128
EPAD = NW * EPW
TRASH = N
NPAD = 10240
RB = 1000
NB = 1024

_mesh = plsc.VectorSubcoreMesh(core_axis_name="c", subcore_axis_name="s")

_sc_params = pltpu.CompilerParams()
if "needs_layout_passes" in pltpu.CompilerParams.__dataclass_fields__:
    _sc_params = dataclasses.replace(_sc_params, needs_layout_passes=False)
_sc_params_untiled = dataclasses.replace(_sc_params, use_tc_tiling_on_sc=False)


def _wid():
    return lax.axis_index("s") * 2 + lax.axis_index("c")


def _take16(v, idx):
    return lax.gather(
        v, idx[:, None],
        lax.GatherDimensionNumbers(offset_dims=(), collapsed_slice_dims=(0,),
                                   start_index_map=(0,)),
        slice_sizes=(1,), mode=lax.GatherScatterMode.PROMISE_IN_BOUNDS)


def _lane_shift(v, k, fill):
    idx = jnp.maximum(lax.iota(jnp.int32, L) - k, 0)
    shifted = _take16(v, idx)
    return jnp.where(lax.iota(jnp.int32, L) >= k, shifted, fill)


def _seg_tail_reduce(keys, vals, is_sum):
    ident = jnp.float32(0.0) if is_sum else jnp.float32(-1e30)
    ks, vs = plsc.sort_key_val(keys, vals)
    for k in (1, 2, 4, 8):
        pk = _lane_shift(ks, k, jnp.int32(-1))
        pv = _lane_shift(vs, k, ident)
        same = pk == ks
        if is_sum:
            vs = vs + jnp.where(same, pv, ident)
        else:
            vs = jnp.maximum(vs, jnp.where(same, pv, ident))
    nxt = _take16(ks, jnp.minimum(lax.iota(jnp.int32, L) + 1, L - 1))
    tail = (lax.iota(jnp.int32, L) == L - 1) | (nxt != ks)
    return ks, vs, tail


def _seg_sum_update(acc_ref, keys, vals):
    ks, vs, tail = _seg_tail_reduce(keys, vals, is_sum=True)
    plsc.addupdate_scatter(acc_ref, [ks], vs, mask=tail)


def _seg_max_update(acc_ref, keys, vals):
    ks, vs, tail = _seg_tail_reduce(keys, vals, is_sum=False)
    cur = plsc.load_gather(acc_ref, [ks], mask=tail)
    plsc.store_scatter(acc_ref, [ks], jnp.maximum(cur, vs), mask=tail)


def _leaky(t):
    return jnp.where(t >= 0, t, 0.2 * t)


@functools.partial(
    pl.kernel, mesh=_mesh, compiler_params=_sc_params,
    out_type=jax.ShapeDtypeStruct((NW, NPAD), jnp.float32),
    scratch_types=[
        pltpu.VMEM((EPW,), jnp.int32),
        pltpu.VMEM((NPAD,), jnp.float32),
        pltpu.SemaphoreType.DMA,
    ],
)
def _sc_deg(dst_hbm, out_hbm, dst_v, acc, sem):
    wid = _wid()
    pltpu.sync_copy(dst_hbm.at[wid], dst_v)

    @pl.loop(0, NPAD, step=L)
    def _(i):
        acc[pl.ds(i, L)] = jnp.zeros((L,), jnp.float32)

    ones = jnp.ones((L,), jnp.float32)

    @pl.loop(0, EPW, step=L)
    def _(i):
        _seg_sum_update(acc, dst_v[pl.ds(i, L)], ones)

    pltpu.sync_copy(acc, out_hbm.at[wid])


@functools.partial(
    pl.kernel, mesh=_mesh, compiler_params=_sc_params,
    out_type=jax.ShapeDtypeStruct((NW, NPAD), jnp.float32),
    scratch_types=[
        pltpu.VMEM((EPW,), jnp.int32),
        pltpu.VMEM((EPW,), jnp.int32),
        pltpu.VMEM((NPAD,), jnp.float32),
        pltpu.VMEM((NPAD,), jnp.float32),
        pltpu.VMEM((NPAD,), jnp.float32),
        pltpu.SemaphoreType.DMA,
    ],
)
def _sc_segmax(src_hbm, dst_hbm, s_hbm, d_hbm, out_hbm,
               src_v, dst_v, s_v, d_v, acc, sem):
    wid = _wid()
    pltpu.sync_copy(src_hbm.at[wid], src_v)
    pltpu.sync_copy(dst_hbm.at[wid], dst_v)
    pltpu.sync_copy(s_hbm, s_v)
    pltpu.sync_copy(d_hbm, d_v)

    @pl.loop(0, NPAD, step=L)
    def _(i):
        acc[pl.ds(i, L)] = jnp.full((L,), -1e30, jnp.float32)

    @pl.loop(0, EPW, step=L)
    def _(i):
        src16 = src_v[pl.ds(i, L)]
        dst16 = dst_v[pl.ds(i, L)]
        alpha = _leaky(plsc.load_gather(s_v, [src16])
                       + plsc.load_gather(d_v, [dst16]))
        _seg_max_update(acc, dst16, alpha)

    pltpu.sync_copy(acc, out_hbm.at[wid])


@functools.partial(
    pl.kernel, mesh=_mesh,
    out_type=jax.ShapeDtypeStruct((2, NPAD, H), jnp.float32),
    scratch_types=[
        pltpu.VMEM((CH, 128), jnp.int32),
        pltpu.VMEM((CH, 128), jnp.int32),
        pltpu.VMEM((128, H), jnp.float32),
        pltpu.VMEM_SHARED((NPAD, H), jnp.float32),
        pltpu.SemaphoreType.DMA,
    ],
)
def _sc_gcn_rows(src_hbm, dst_hbm, tab_hbm, z_hbm, out_hbm,
                 src_v, dst_v, rows, acc_sh, sem):
    core = lax.axis_index("c")
    sid = lax.axis_index("s")
    wid = _wid()
    pltpu.sync_copy(src_hbm.at[wid], src_v)
    pltpu.sync_copy(dst_hbm.at[wid], dst_v)

    sl = pl.ds(sid * (NPAD // 16), NPAD // 16)
    pltpu.sync_copy(z_hbm.at[sl], acc_sh.at[sl])
    plsc.subcore_barrier()

    @pl.loop(0, CH)
    def _(i):
        pltpu.async_copy(tab_hbm.at[src_v.at[i]], rows, sem).wait()
        pltpu.sync_copy(rows, acc_sh.at[dst_v.at[i]], add=True)

    plsc.subcore_barrier()
    pltpu.sync_copy(acc_sh.at[sl], out_hbm.at[core, sl])


H2 = H // 2
EC = 64
CH2 = EPAD // 16 // EC


@functools.partial(
    pl.kernel, mesh=_mesh, compiler_params=_sc_params_untiled,
    out_type=(jax.ShapeDtypeStruct((2, NPAD, H2), jnp.float32),
              jax.ShapeDtypeStruct((16, NPAD), jnp.float32)),
    scratch_types=[
        pltpu.VMEM((CH2, EC), jnp.int32),
        pltpu.VMEM((CH2, EC), jnp.int32),
        pltpu.VMEM((NPAD,), jnp.float32),
        pltpu.VMEM((NPAD,), jnp.float32),
        pltpu.VMEM((NPAD,), jnp.float32),
        pltpu.VMEM((NPAD,), jnp.float32),
        pltpu.VMEM((EC,), jnp.float32),
        pltpu.VMEM((EC, H2), jnp.float32),
        pltpu.VMEM_SHARED((NPAD, H2), jnp.float32),
        pltpu.SemaphoreType.DMA,
    ],
)
def _sc_gat_rows(src_hbm, dst_hbm, tab_hbm, s_hbm, d_hbm, m_hbm, z_hbm,
                 out_hbm, den_hbm,
                 src_v, dst_v, s_v, d_v, m_v, den_acc, w_buf, rows,
                 acc_sh, sem):
    core = lax.axis_index("c")
    sid = lax.axis_index("s")
    pltpu.sync_copy(src_hbm.at[sid], src_v)
    pltpu.sync_copy(dst_hbm.at[sid], dst_v)
    pltpu.sync_copy(s_hbm, s_v)
    pltpu.sync_copy(d_hbm, d_v)
    pltpu.sync_copy(m_hbm, m_v)

    @pl.loop(0, NPAD, step=L)
    def _(i):
        den_acc[pl.ds(i, L)] = jnp.zeros((L,), jnp.float32)

    sl = pl.ds(sid * (NPAD // 16), NPAD // 16)
    pltpu.sync_copy(z_hbm.at[sl], acc_sh.at[sl])
    plsc.subcore_barrier()

    @pl.loop(0, CH2)
    def _(i):
        cp = pltpu.async_copy(tab_hbm.at[core].at[src_v.at[i]], rows, sem)
        exs = []
        dsts = []
        for j in range(EC // L):
            src16 = src_v[i, pl.ds(j * L, L)]
            dst16 = dst_v[i, pl.ds(j * L, L)]
            alpha = _leaky(plsc.load_gather(s_v, [src16])
                           + plsc.load_gather(d_v, [dst16]))
            ex = jnp.exp(alpha - plsc.load_gather(m_v, [dst16]))
            w_buf[pl.ds(j * L, L)] = ex
            exs.append(ex)
            dsts.append(dst16)

        @pl.when(core == 0)
        def _():
            for dst16, ex in zip(dsts, exs):
                _seg_sum_update(den_acc, dst16, ex)

        cp.wait()

        @pl.loop(0, EC, step=L)
        def _(g):
            wv = w_buf[pl.ds(g, L)]
            for l in range(L):
                w = wv[l]
                for c in range(H2 // L):
                    rows[g + l, pl.ds(c * L, L)] = (
                        rows[g + l, pl.ds(c * L, L)] * w)

        pltpu.sync_copy(rows, acc_sh.at[dst_v.at[i]], add=True)

    plsc.subcore_barrier()
    pltpu.sync_copy(acc_sh.at[sl], out_hbm.at[core, sl])

    @pl.when(core == 0)
    def _():
        pltpu.sync_copy(den_acc, den_hbm.at[sid])


def _row_spec(b, cols=None):
    if cols is None:
        return pl.BlockSpec((b,), lambda i: (i,))
    return pl.BlockSpec((b, cols), lambda i: (i, 0))


def _full_spec(shape):
    nd = len(shape)
    return pl.BlockSpec(shape, lambda i: (0,) * nd)


def _tc_pre(x_pad, Wg0, degP):
    def body(x_ref, w_ref, deg_ref, g_ref, dinv_ref):
        deg = jnp.sum(deg_ref[...], axis=0)
        dinv = jnp.where(deg > 0, lax.rsqrt(deg), 0.0)
        hw = jnp.dot(x_ref[...], w_ref[...],
                     preferred_element_type=jnp.float32)
        g_ref[...] = hw * dinv[:, None]
        dinv_ref[...] = dinv

    return pl.pallas_call(
        body,
        grid=(NPAD // NB,),
        in_specs=[_row_spec(NB, F_IN), _full_spec((F_IN, H)),
                  pl.BlockSpec((NW, NB), lambda i: (0, i))],
        out_specs=[_row_spec(NB, H), _row_spec(NB)],
        out_shape=[jax.ShapeDtypeStruct((NPAD, H), jnp.float32),
                   jax.ShapeDtypeStruct((NPAD,), jnp.float32)],
    )(x_pad, Wg0, degP)


def _tc_gcn_mid(rowP, dinv, b, W, scale_out):
    def body(p_ref, dinv_ref, b_ref, w_ref, g_ref):
        agg = p_ref[0] + p_ref[1]
        dinv_v = dinv_ref[...]
        h = jnp.maximum(agg * dinv_v[:, None] + b_ref[...][None, :], 0.0)
        hw = jnp.dot(h, w_ref[...], preferred_element_type=jnp.float32)
        g_ref[...] = hw * dinv_v[:, None] if scale_out else hw

    return pl.pallas_call(
        body,
        grid=(NPAD // NB,),
        in_specs=[pl.BlockSpec((2, NB, H), lambda i: (0, i, 0)),
                  _row_spec(NB), _full_spec((H,)), _full_spec((H, H))],
        out_specs=_row_spec(NB, H),
        out_shape=jax.ShapeDtypeStruct((NPAD, H), jnp.float32),
    )(rowP, dinv, b, W)


def _tc_logits(hw, a_s, a_d):
    def body(hw_ref, as_ref, ad_ref, s_ref, d_ref):
        hwv = hw_ref[...]
        s_ref[...] = jnp.sum(hwv * as_ref[...][None, :], axis=1)
        d_ref[...] = jnp.sum(hwv * ad_ref[...][None, :], axis=1)

    return pl.pallas_call(
        body,
        grid=(NPAD // NB,),
        in_specs=[_row_spec(NB, H), _full_spec((H,)), _full_spec((H,))],
        out_specs=[_row_spec(NB), _row_spec(NB)],
        out_shape=[jax.ShapeDtypeStruct((NPAD,), jnp.float32),
                   jax.ShapeDtypeStruct((NPAD,), jnp.float32)],
    )(hw, a_s, a_d)


def _tc_max_combine(mP):
    def body(m_ref, o_ref):
        o_ref[...] = jnp.max(m_ref[...], axis=0)

    return pl.pallas_call(
        body,
        grid=(NPAD // NB,),
        in_specs=[pl.BlockSpec((NW, NB), lambda i: (0, i))],
        out_specs=_row_spec(NB),
        out_shape=jax.ShapeDtypeStruct((NPAD,), jnp.float32),
    )(mP)


def _tc_gat_mid(rowP, denP, b, W):
    def body(p_ref, den_ref, b_ref, w_ref, g_ref):
        denom = jnp.sum(den_ref[...], axis=0)
        agg = (jnp.concatenate([p_ref[0], p_ref[1]], axis=1)
               / (denom + 1e-16)[:, None])
        h = jnp.maximum(agg + b_ref[...][None, :], 0.0)
        g_ref[...] = jnp.dot(h, w_ref[...],
                             preferred_element_type=jnp.float32)

    return pl.pallas_call(
        body,
        grid=(NPAD // NB,),
        in_specs=[pl.BlockSpec((2, NB, H2), lambda i: (0, i, 0)),
                  pl.BlockSpec((16, NB), lambda i: (0, i)),
                  _full_spec((H,)), _full_spec((H, H))],
        out_specs=_row_spec(NB, H),
        out_shape=jax.ShapeDtypeStruct((NPAD, H), jnp.float32),
    )(rowP, denP, b, W)


def _tc_final(rowP, denP, ba, Wl, bl):
    def body(p_ref, den_ref, ba_ref, w_ref, bl_ref, o_ref):
        denom = jnp.sum(den_ref[...], axis=0)
        agg = (jnp.concatenate([p_ref[0], p_ref[1]], axis=1)
               / (denom + 1e-16)[:, None])
        h = jnp.maximum(agg + ba_ref[...][None, :], 0.0)
        o_ref[...] = (jnp.dot(h, w_ref[...],
                              preferred_element_type=jnp.float32)
                      + bl_ref[...][None, :])

    return pl.pallas_call(
        body,
        grid=(NPAD // NB,),
        in_specs=[pl.BlockSpec((2, NB, H2), lambda i: (0, i, 0)),
                  pl.BlockSpec((16, NB), lambda i: (0, i)),
                  _full_spec((H,)), _full_spec((H, C)), _full_spec((C,))],
        out_specs=pl.BlockSpec((NB, C), lambda i: (i, 0)),
        out_shape=jax.ShapeDtypeStruct((NPAD, C), jnp.float32),
    )(rowP, denP, ba, Wl, bl)


def kernel(x, edge_index, Wg0, bg0, Wg1, bg1, Wa0, as0, ad0, ba0,
           Wa1, as1, ad1, ba1, Wl, bl):
    padn = EPAD - (E + N)
    loop = jnp.arange(N, dtype=jnp.int32)
    src = jnp.concatenate(
        [edge_index[0], loop, jnp.arange(padn, dtype=jnp.int32) % N])
    dst = jnp.concatenate(
        [edge_index[1], loop, jnp.full((padn,), TRASH, jnp.int32)])
    src2 = src.reshape(NW, EPW)
    dst2 = dst.reshape(NW, EPW)
    src3 = src.reshape(NW, CH, 128)
    dst3 = dst.reshape(NW, CH, 128)
    src4 = src.reshape(16, CH2, EC)
    dst4 = dst.reshape(16, CH2, EC)
    x_pad = jnp.pad(x, ((0, NPAD - N), (0, 0)))
    zrows = jnp.zeros((NPAD, H), jnp.float32)
    zrows2 = jnp.zeros((NPAD, H2), jnp.float32)

    degP = _sc_deg(dst2)

    g0, dinv = _tc_pre(x_pad, Wg0, degP)
    rowP1 = _sc_gcn_rows(src3, dst3, g0, zrows)
    g1 = _tc_gcn_mid(rowP1, dinv, bg0, Wg1, scale_out=True)
    rowP2 = _sc_gcn_rows(src3, dst3, g1, zrows)
    hw2 = _tc_gcn_mid(rowP2, dinv, bg1, Wa0, scale_out=False)
    s2, d2 = _tc_logits(hw2, as0, ad0)
    m2 = _tc_max_combine(_sc_segmax(src2, dst2, s2, d2))
    tab2 = jnp.stack([hw2[:, :H2], hw2[:, H2:]])
    rowP3, denP3 = _sc_gat_rows(src4, dst4, tab2, s2, d2, m2, zrows2)
    hw3 = _tc_gat_mid(rowP3, denP3, ba0, Wa1)
    s3, d3 = _tc_logits(hw3, as1, ad1)
    m3 = _tc_max_combine(_sc_segmax(src2, dst2, s3, d3))
    tab3 = jnp.stack([hw3[:, :H2], hw3[:, H2:]])
    rowP4, denP4 = _sc_gat_rows(src4, dst4, tab3, s3, d3, m3, zrows2)
    return _tc_final(rowP4, denP4, ba1, Wl, bl)[:N]
